# Initial kernel scaffold; baseline (speedup 1.0000x reference)
import jax
import jax.numpy as jnp
from jax.experimental import pallas as pl


def kernel(indices, weight):
    raise NotImplementedError("write your pallas kernel here")


# speedup vs baseline: 78.2227x; 78.2227x over previous
import functools

import jax
import jax.numpy as jnp
from jax import lax
from jax.experimental import pallas as pl
from jax.experimental.pallas import tpu as pltpu
from jax.experimental.pallas import tpu_sc as plsc

DIM = 64
NC, NS = 2, 16
NW = NC * NS
B = 4096 * 200
BPW = B // NW
CHUNK = 1024
K = CHUNK // 128
NCHUNK = BPW // CHUNK
IDX_ROWS_PW = BPW // 128

_mesh = plsc.VectorSubcoreMesh(
    core_axis_name="c", subcore_axis_name="s", num_cores=NC, num_subcores=NS)


@functools.partial(
    pl.kernel,
    out_type=jax.ShapeDtypeStruct((B, DIM), jnp.float32),
    mesh=_mesh,
    scratch_types=[
        pltpu.VMEM((IDX_ROWS_PW, 128), jnp.int32),
        pltpu.VMEM((CHUNK, DIM), jnp.float32),
        pltpu.SemaphoreType.DMA,
    ],
    compiler_params=pltpu.CompilerParams(use_tc_tiling_on_sc=False),
)
def _gather(idx_hbm, tab_hbm, out_hbm, idx_v, rows_v, sem):
    wid = lax.axis_index("s") * NC + lax.axis_index("c")
    base = wid * BPW
    row_base = wid * IDX_ROWS_PW
    pltpu.sync_copy(idx_hbm.at[pl.ds(row_base, IDX_ROWS_PW)], idx_v)

    @pl.loop(0, NCHUNK)
    def _chunk(g):
        copies = [
            pltpu.async_copy(
                tab_hbm.at[idx_v.at[g * K + j]],
                rows_v.at[pl.ds(j * 128, 128)],
                sem)
            for j in range(K)
        ]
        for c in copies:
            c.wait()
        pltpu.sync_copy(rows_v, out_hbm.at[pl.ds(base + g * CHUNK, CHUNK)])


def kernel(indices, weight):
    idx2d = indices.reshape(B // 128, 128)
    out = _gather(idx2d, weight)
    return out.reshape(indices.shape + (DIM,))


# speedup vs baseline: 78.8299x; 1.0078x over previous
import functools

import jax
import jax.numpy as jnp
from jax import lax
from jax.experimental import pallas as pl
from jax.experimental.pallas import tpu as pltpu
from jax.experimental.pallas import tpu_sc as plsc

DIM = 64
NC, NS = 2, 16
NW = NC * NS
B = 4096 * 200
BPW = B // NW
CHUNK = 800
NCHUNK = BPW // CHUNK

_mesh = plsc.VectorSubcoreMesh(
    core_axis_name="c", subcore_axis_name="s", num_cores=NC, num_subcores=NS)


@functools.partial(
    pl.kernel,
    out_type=jax.ShapeDtypeStruct((B, DIM), jnp.float32),
    mesh=_mesh,
    scratch_types=[
        pltpu.VMEM((BPW,), jnp.int32),
        pltpu.VMEM((CHUNK, DIM), jnp.float32),
        pltpu.VMEM((CHUNK, DIM), jnp.float32),
        pltpu.SemaphoreType.DMA,
        pltpu.SemaphoreType.DMA,
        pltpu.SemaphoreType.DMA,
        pltpu.SemaphoreType.DMA,
    ],
    compiler_params=pltpu.CompilerParams(use_tc_tiling_on_sc=False),
)
def _gather(idx_hbm, tab_hbm, out_hbm, idx_v, rows0, rows1,
            gsem0, gsem1, ssem0, ssem1):
    wid = lax.axis_index("s") * NC + lax.axis_index("c")
    base = wid * BPW
    pltpu.sync_copy(idx_hbm.at[pl.ds(base, BPW)], idx_v)

    def idx_sl(g):
        return idx_v.at[pl.ds(g * CHUNK, CHUNK)]

    def out_sl(g):
        return out_hbm.at[pl.ds(base + g * CHUNK, CHUNK)]

    pltpu.async_copy(tab_hbm.at[idx_sl(0)], rows0, gsem0)

    @pl.loop(0, NCHUNK, step=2)
    def _pair(g):
        pltpu.make_async_copy(tab_hbm.at[idx_sl(g)], rows0, gsem0).wait()
        pltpu.async_copy(tab_hbm.at[idx_sl(g + 1)], rows1, gsem1)
        st0 = pltpu.async_copy(rows0, out_sl(g), ssem0)
        st0.wait()

        @pl.when(g + 2 < NCHUNK)
        def _():
            pltpu.async_copy(tab_hbm.at[idx_sl(g + 2)], rows0, gsem0)

        pltpu.make_async_copy(tab_hbm.at[idx_sl(g + 1)], rows1, gsem1).wait()
        st1 = pltpu.async_copy(rows1, out_sl(g + 1), ssem1)
        st1.wait()


def kernel(indices, weight):
    out = _gather(indices.reshape(B), weight)
    return out.reshape(indices.shape + (DIM,))
